# baseline (speedup 1.0000x reference)
import functools

import jax
import jax.numpy as jnp
import numpy as np
from jax import lax
from jax.experimental import pallas as pl
from jax.experimental.pallas import tpu as pltpu
from jax.experimental.pallas import tpu_sc as plsc

_NUM_CORES = 2
_NUM_SUBCORES = 16
_LANES = 16
_NUM_WORKERS = _NUM_CORES * _NUM_SUBCORES

_B, _D, _G = 16384, 11, 10
_ROWS = _B // _NUM_WORKERS
_STEPS = _ROWS // _LANES

_T2_BITS = (0x3C23D70A, 0x3D23D70A, 0x3DB851EC, 0x3E23D70A, 0x3E800000,
            0x3EB851EC, 0x3EFAE146, 0x3F23D70A, 0x3F4F5C2A)
_T2 = tuple(float(np.uint32(b).view(np.float32)) for b in _T2_BITS)


def _vfrho_body(z1_hbm, z2_hbm, dg_hbm, out_hbm, z1_v, z2_v, dg_v, out_v):
    wid = lax.axis_index("s") * _NUM_CORES + lax.axis_index("c")
    base = wid * _ROWS
    pltpu.sync_copy(z1_hbm.at[pl.ds(base * _D, _ROWS * _D)], z1_v)
    pltpu.sync_copy(z2_hbm.at[pl.ds(base * _D, _ROWS * _D)], z2_v)
    pltpu.sync_copy(dg_hbm.at[pl.ds(base * _G, _ROWS * _G)], dg_v)

    lane = lax.iota(jnp.int32, _LANES)
    t2 = [jnp.full((_LANES,), v, jnp.float32) for v in _T2]

    def step(i, carry):
        rows = lane + i * _LANES
        zoff = rows * _D
        x1 = plsc.load_gather(z1_v, [zoff])
        x2 = plsc.load_gather(z2_v, [zoff])
        y1 = plsc.load_gather(z1_v, [zoff + 2])
        y2 = plsc.load_gather(z2_v, [zoff + 2])
        dx = x2 - x1
        dy = y2 - y1
        r2 = dx * dx + dy * dy
        bucket = jnp.zeros((_LANES,), jnp.int32)
        for c in t2:
            bucket = bucket + (r2 >= c).astype(jnp.int32)
        val = plsc.load_gather(dg_v, [rows * _G + bucket])
        out_v[pl.ds(i * _LANES, _LANES)] = val
        return carry

    lax.fori_loop(0, _STEPS, step, 0)
    pltpu.sync_copy(out_v, out_hbm.at[pl.ds(base, _ROWS)])


_vfrho_sc = functools.partial(
    pl.kernel,
    out_type=jax.ShapeDtypeStruct((_B,), jnp.float32),
    mesh=plsc.VectorSubcoreMesh(core_axis_name="c", subcore_axis_name="s"),
    compiler_params=pltpu.CompilerParams(
        needs_layout_passes=False,
        skip_device_barrier=True,
        disable_bounds_checks=True,
        disable_semaphore_checks=True,
    ),
    scratch_types=[
        pltpu.VMEM((_ROWS * _D,), jnp.float32),
        pltpu.VMEM((_ROWS * _D,), jnp.float32),
        pltpu.VMEM((_ROWS * _G,), jnp.float32),
        pltpu.VMEM((_ROWS,), jnp.float32),
    ],
)(_vfrho_body)


def kernel(z_1, z_2, dist_grade):
    return _vfrho_sc(z_1.reshape(-1), z_2.reshape(-1), dist_grade.reshape(-1))


# speedup vs baseline: 2.2028x; 2.2028x over previous
import functools

import jax
import jax.numpy as jnp
import numpy as np
from jax import lax
from jax.experimental import pallas as pl
from jax.experimental.pallas import tpu as pltpu
from jax.experimental.pallas import tpu_sc as plsc

_NUM_CORES = 2
_NUM_SUBCORES = 16
_LANES = 16
_NUM_WORKERS = _NUM_CORES * _NUM_SUBCORES

_B, _D, _G = 16384, 11, 10
_ROWS = _B // _NUM_WORKERS
_STEPS = _ROWS // _LANES

_T2_BITS = (0x3C23D70A, 0x3D23D70A, 0x3DB851EC, 0x3E23D70A, 0x3E800000,
            0x3EB851EC, 0x3EFAE146, 0x3F23D70A, 0x3F4F5C2A)
_T2 = tuple(float(np.uint32(b).view(np.float32)) for b in _T2_BITS)


def _vfrho_body(x1_hbm, y1_hbm, x2_hbm, y2_hbm, dg_hbm, out_hbm,
                x1_v, y1_v, x2_v, y2_v, dg_v, out_v, sem):
    wid = lax.axis_index("s") * _NUM_CORES + lax.axis_index("c")
    base = wid * _ROWS
    chunk = pl.ds(base, _ROWS)

    copies = [
        pltpu.async_copy(x1_hbm.at[chunk], x1_v, sem),
        pltpu.async_copy(y1_hbm.at[chunk], y1_v, sem),
        pltpu.async_copy(x2_hbm.at[chunk], x2_v, sem),
        pltpu.async_copy(y2_hbm.at[chunk], y2_v, sem),
    ]
    for g in range(_G):
        copies.append(pltpu.async_copy(
            dg_hbm[g].at[chunk], dg_v.at[pl.ds(g * _ROWS, _ROWS)], sem))
    for c in copies:
        c.wait()

    lane = lax.iota(jnp.int32, _LANES)
    t2 = [jnp.full((_LANES,), v, jnp.float32) for v in _T2]

    def step(i, carry):
        sl = pl.ds(i * _LANES, _LANES)
        dx = x2_v[sl] - x1_v[sl]
        dy = y2_v[sl] - y1_v[sl]
        r2 = dx * dx + dy * dy
        bucket = jnp.zeros((_LANES,), jnp.int32)
        for c in t2:
            bucket = bucket + (r2 >= c).astype(jnp.int32)
        rows = lane + i * _LANES
        val = plsc.load_gather(dg_v, [bucket * _ROWS + rows])
        out_v[sl] = val
        return carry

    lax.fori_loop(0, _STEPS, step, 0)
    pltpu.sync_copy(out_v, out_hbm.at[chunk])


_vfrho_sc = functools.partial(
    pl.kernel,
    out_type=jax.ShapeDtypeStruct((_B,), jnp.float32),
    mesh=plsc.VectorSubcoreMesh(core_axis_name="c", subcore_axis_name="s"),
    compiler_params=pltpu.CompilerParams(needs_layout_passes=False),
    scratch_types=[
        pltpu.VMEM((_ROWS,), jnp.float32),
        pltpu.VMEM((_ROWS,), jnp.float32),
        pltpu.VMEM((_ROWS,), jnp.float32),
        pltpu.VMEM((_ROWS,), jnp.float32),
        pltpu.VMEM((_G * _ROWS,), jnp.float32),
        pltpu.VMEM((_ROWS,), jnp.float32),
        pltpu.SemaphoreType.DMA,
    ],
)(_vfrho_body)


def kernel(z_1, z_2, dist_grade):
    x1 = z_1[:, 0]
    y1 = z_1[:, 2]
    x2 = z_2[:, 0]
    y2 = z_2[:, 2]
    dg_cols = [dist_grade[:, g] for g in range(_G)]
    return _vfrho_sc(x1, y1, x2, y2, dg_cols)


# speedup vs baseline: 2.2749x; 1.0327x over previous
import functools

import jax
import jax.numpy as jnp
import numpy as np
from jax import lax
from jax.experimental import pallas as pl
from jax.experimental.pallas import tpu as pltpu
from jax.experimental.pallas import tpu_sc as plsc

_NUM_CORES = 2
_NUM_SUBCORES = 16
_LANES = 16
_NUM_WORKERS = _NUM_CORES * _NUM_SUBCORES

_B, _D, _G = 16384, 11, 10
_ROWS = _B // _NUM_WORKERS
_STEPS = _ROWS // _LANES

_T2_BITS = (0x3C23D70A, 0x3D23D70A, 0x3DB851EC, 0x3E23D70A, 0x3E800000,
            0x3EB851EC, 0x3EFAE146, 0x3F23D70A, 0x3F4F5C2A)
_T2 = tuple(float(np.uint32(b).view(np.float32)) for b in _T2_BITS)


def _vfrho_body(cols_hbm, out_hbm,
                x1_v, y1_v, x2_v, y2_v, dg_v, out_v, sem):
    wid = lax.axis_index("s") * _NUM_CORES + lax.axis_index("c")
    base = wid * _ROWS
    chunk = pl.ds(base, _ROWS)

    copies = [
        pltpu.async_copy(cols_hbm.at[pl.ds(0 * _B + base, _ROWS)], x1_v, sem),
        pltpu.async_copy(cols_hbm.at[pl.ds(1 * _B + base, _ROWS)], y1_v, sem),
        pltpu.async_copy(cols_hbm.at[pl.ds(2 * _B + base, _ROWS)], x2_v, sem),
        pltpu.async_copy(cols_hbm.at[pl.ds(3 * _B + base, _ROWS)], y2_v, sem),
    ]
    for g in range(_G):
        copies.append(pltpu.async_copy(
            cols_hbm.at[pl.ds((4 + g) * _B + base, _ROWS)],
            dg_v.at[pl.ds(g * _ROWS, _ROWS)], sem))
    for c in copies:
        c.wait()

    lane = lax.iota(jnp.int32, _LANES)
    t2 = [jnp.full((_LANES,), v, jnp.float32) for v in _T2]

    def step(i, carry):
        sl = pl.ds(i * _LANES, _LANES)
        dx = x2_v[sl] - x1_v[sl]
        dy = y2_v[sl] - y1_v[sl]
        r2 = dx * dx + dy * dy
        bucket = jnp.zeros((_LANES,), jnp.int32)
        for c in t2:
            bucket = bucket + (r2 >= c).astype(jnp.int32)
        rows = lane + i * _LANES
        val = plsc.load_gather(dg_v, [bucket * _ROWS + rows])
        out_v[sl] = val
        return carry

    lax.fori_loop(0, _STEPS, step, 0)
    pltpu.sync_copy(out_v, out_hbm.at[chunk])


_vfrho_sc = functools.partial(
    pl.kernel,
    out_type=jax.ShapeDtypeStruct((_B,), jnp.float32),
    mesh=plsc.VectorSubcoreMesh(core_axis_name="c", subcore_axis_name="s"),
    compiler_params=pltpu.CompilerParams(needs_layout_passes=False),
    scratch_types=[
        pltpu.VMEM((_ROWS,), jnp.float32),
        pltpu.VMEM((_ROWS,), jnp.float32),
        pltpu.VMEM((_ROWS,), jnp.float32),
        pltpu.VMEM((_ROWS,), jnp.float32),
        pltpu.VMEM((_G * _ROWS,), jnp.float32),
        pltpu.VMEM((_ROWS,), jnp.float32),
        pltpu.SemaphoreType.DMA,
    ],
)(_vfrho_body)


def kernel(z_1, z_2, dist_grade):
    cols = jnp.concatenate(
        [z_1[:, 0], z_1[:, 2], z_2[:, 0], z_2[:, 2]]
        + [dist_grade[:, g] for g in range(_G)])
    return _vfrho_sc(cols)


# speedup vs baseline: 2.4084x; 1.0587x over previous
import functools

import jax
import jax.numpy as jnp
import numpy as np
from jax import lax
from jax.experimental import pallas as pl
from jax.experimental.pallas import tpu as pltpu
from jax.experimental.pallas import tpu_sc as plsc

_NUM_CORES = 2
_NUM_SUBCORES = 16
_LANES = 16
_NUM_WORKERS = _NUM_CORES * _NUM_SUBCORES

_B, _D, _G = 16384, 11, 10
_ROWS = _B // _NUM_WORKERS
_STEPS = _ROWS // _LANES

_T2_BITS = (0x3C23D70A, 0x3D23D70A, 0x3DB851EC, 0x3E23D70A, 0x3E800000,
            0x3EB851EC, 0x3EFAE146, 0x3F23D70A, 0x3F4F5C2A)
_T2 = tuple(float(np.uint32(b).view(np.float32)) for b in _T2_BITS)


def _vfrho_body(x1_hbm, y1_hbm, x2_hbm, y2_hbm, dgt_hbm, out_hbm,
                x1_v, y1_v, x2_v, y2_v, dg_v, out_v, sem):
    wid = lax.axis_index("s") * _NUM_CORES + lax.axis_index("c")
    base = wid * _ROWS
    chunk = pl.ds(base, _ROWS)

    copies = [
        pltpu.async_copy(x1_hbm.at[chunk], x1_v, sem),
        pltpu.async_copy(y1_hbm.at[chunk], y1_v, sem),
        pltpu.async_copy(x2_hbm.at[chunk], x2_v, sem),
        pltpu.async_copy(y2_hbm.at[chunk], y2_v, sem),
    ]
    for g in range(_G):
        copies.append(pltpu.async_copy(
            dgt_hbm.at[pl.ds(g * _B + base, _ROWS)],
            dg_v.at[pl.ds(g * _ROWS, _ROWS)], sem))
    for c in copies:
        c.wait()

    lane = lax.iota(jnp.int32, _LANES)
    t2 = [jnp.full((_LANES,), v, jnp.float32) for v in _T2]

    def step(i, carry):
        sl = pl.ds(i * _LANES, _LANES)
        dx = x2_v[sl] - x1_v[sl]
        dy = y2_v[sl] - y1_v[sl]
        r2 = dx * dx + dy * dy
        bucket = jnp.zeros((_LANES,), jnp.int32)
        for c in t2:
            bucket = bucket + (r2 >= c).astype(jnp.int32)
        rows = lane + i * _LANES
        val = plsc.load_gather(dg_v, [bucket * _ROWS + rows])
        out_v[sl] = val
        return carry

    lax.fori_loop(0, _STEPS, step, 0)
    pltpu.sync_copy(out_v, out_hbm.at[chunk])


_vfrho_sc = functools.partial(
    pl.kernel,
    out_type=jax.ShapeDtypeStruct((_B,), jnp.float32),
    mesh=plsc.VectorSubcoreMesh(core_axis_name="c", subcore_axis_name="s"),
    compiler_params=pltpu.CompilerParams(needs_layout_passes=False),
    scratch_types=[
        pltpu.VMEM((_ROWS,), jnp.float32),
        pltpu.VMEM((_ROWS,), jnp.float32),
        pltpu.VMEM((_ROWS,), jnp.float32),
        pltpu.VMEM((_ROWS,), jnp.float32),
        pltpu.VMEM((_G * _ROWS,), jnp.float32),
        pltpu.VMEM((_ROWS,), jnp.float32),
        pltpu.SemaphoreType.DMA,
    ],
)(_vfrho_body)


def kernel(z_1, z_2, dist_grade):
    dgt = dist_grade.T.reshape(-1)
    return _vfrho_sc(z_1[:, 0], z_1[:, 2], z_2[:, 0], z_2[:, 2], dgt)


# speedup vs baseline: 2.4101x; 1.0007x over previous
import functools

import jax
import jax.numpy as jnp
import numpy as np
from jax import lax
from jax.experimental import pallas as pl
from jax.experimental.pallas import tpu as pltpu
from jax.experimental.pallas import tpu_sc as plsc

_NUM_CORES = 2
_NUM_SUBCORES = 16
_LANES = 16
_NUM_WORKERS = _NUM_CORES * _NUM_SUBCORES

_B, _D, _G = 16384, 11, 10
_ROWS = _B // _NUM_WORKERS
_STEPS = _ROWS // _LANES

_T2_BITS = (0x3C23D70A, 0x3D23D70A, 0x3DB851EC, 0x3E23D70A, 0x3E800000,
            0x3EB851EC, 0x3EFAE146, 0x3F23D70A, 0x3F4F5C2A)
_T2 = tuple(float(np.uint32(b).view(np.float32)) for b in _T2_BITS)


def _vfrho_body(x1_hbm, y1_hbm, x2_hbm, y2_hbm, dgt_hbm, out_hbm,
                x1_v, y1_v, x2_v, y2_v, dg_v, idx_v, out_v, zsem, gsem):
    wid = lax.axis_index("s") * _NUM_CORES + lax.axis_index("c")
    base = wid * _ROWS
    chunk = pl.ds(base, _ROWS)

    z_copies = [
        pltpu.async_copy(x1_hbm.at[chunk], x1_v, zsem),
        pltpu.async_copy(y1_hbm.at[chunk], y1_v, zsem),
        pltpu.async_copy(x2_hbm.at[chunk], x2_v, zsem),
        pltpu.async_copy(y2_hbm.at[chunk], y2_v, zsem),
    ]
    def issue_dg(g, carry):
        pltpu.make_async_copy(
            dgt_hbm.at[pl.ds(g * _B + base, _ROWS)],
            dg_v.at[pl.ds(g * _ROWS, _ROWS)], gsem).start()
        return carry
    lax.fori_loop(0, _G, issue_dg, 0)
    for c in z_copies:
        c.wait()

    lane = lax.iota(jnp.int32, _LANES)
    t2 = [jnp.full((_LANES,), v, jnp.float32) for v in _T2]

    def step_bucket(i, carry):
        sl = pl.ds(i * _LANES, _LANES)
        dx = x2_v[sl] - x1_v[sl]
        dy = y2_v[sl] - y1_v[sl]
        r2 = dx * dx + dy * dy
        bucket = jnp.zeros((_LANES,), jnp.int32)
        for c in t2:
            bucket = bucket + (r2 >= c).astype(jnp.int32)
        idx_v[sl] = bucket * _ROWS + (lane + i * _LANES)
        return carry
    lax.fori_loop(0, _STEPS, step_bucket, 0)

    pltpu.make_async_copy(dgt_hbm.at[pl.ds(0, _G * _ROWS)], dg_v, gsem).wait()

    def step_gather(i, carry):
        sl = pl.ds(i * _LANES, _LANES)
        out_v[sl] = plsc.load_gather(dg_v, [idx_v[sl]])
        return carry
    lax.fori_loop(0, _STEPS, step_gather, 0)
    pltpu.sync_copy(out_v, out_hbm.at[chunk])


_vfrho_sc = functools.partial(
    pl.kernel,
    out_type=jax.ShapeDtypeStruct((_B,), jnp.float32),
    mesh=plsc.VectorSubcoreMesh(core_axis_name="c", subcore_axis_name="s"),
    compiler_params=pltpu.CompilerParams(needs_layout_passes=False),
    scratch_types=[
        pltpu.VMEM((_ROWS,), jnp.float32),
        pltpu.VMEM((_ROWS,), jnp.float32),
        pltpu.VMEM((_ROWS,), jnp.float32),
        pltpu.VMEM((_ROWS,), jnp.float32),
        pltpu.VMEM((_G * _ROWS,), jnp.float32),
        pltpu.VMEM((_ROWS,), jnp.int32),
        pltpu.VMEM((_ROWS,), jnp.float32),
        pltpu.SemaphoreType.DMA,
        pltpu.SemaphoreType.DMA,
    ],
)(_vfrho_body)


def kernel(z_1, z_2, dist_grade):
    dgt = dist_grade.T.reshape(-1)
    return _vfrho_sc(z_1[:, 0], z_1[:, 2], z_2[:, 0], z_2[:, 2], dgt)
